# baseline (speedup 1.0000x reference)
import functools

import jax
import jax.numpy as jnp
from jax import lax
from jax.experimental import pallas as pl
from jax.experimental.pallas import tpu as pltpu
from jax.experimental.pallas import tpu_sc as plsc

_NC = 2
_NS = 16
_NW = _NC * _NS
_L = 16


def kernel(x):
    n_rows, n_cols = x.shape
    n = n_rows * n_cols
    per_w = n // _NW
    mesh = plsc.VectorSubcoreMesh(core_axis_name="c", subcore_axis_name="s")

    @functools.partial(
        pl.kernel,
        mesh=mesh,
        out_type=jax.ShapeDtypeStruct((n,), jnp.float32),
        scratch_types=[pltpu.VMEM((per_w,), jnp.float32)],
    )
    def sc_mask(x_hbm, out_hbm, buf):
        wid = lax.axis_index("s") * _NC + lax.axis_index("c")
        base = wid * per_w
        pltpu.sync_copy(x_hbm.at[pl.ds(base, per_w)], buf)

        @plsc.parallel_loop(0, per_w, step=_L, unroll=8)
        def _(i):
            v = buf[pl.ds(i, _L)]
            buf[pl.ds(i, _L)] = jnp.where(v > 0, v, 0.0)

        pltpu.sync_copy(buf, out_hbm.at[pl.ds(base, per_w)])

    return sc_mask(x.reshape(n)).reshape(n_rows, n_cols)
